# baseline (speedup 1.0000x reference)
import functools

import jax
import jax.numpy as jnp
from jax import lax
from jax.experimental import pallas as pl
from jax.experimental.pallas import tpu as pltpu
from jax.experimental.pallas import tpu_sc as plsc

B = 16384
K = 32
NC = 2
NS = 16
NW = NC * NS
BPW = B // NW
RB = 16
NE = BPW * K
NG = BPW // RB
CH = 128
CPW = 8
EPW = CH * CPW
NWAVE = NE // EPW
KPWAVE = EPW // BPW

UROWS, VROWS = 1000000, 100000
UBT = 651
UCB = 13
VBT = 71
VCB = 12
UGS = UCB * UBT * 1024
VGS = VCB * VBT * 1024

_mesh = plsc.VectorSubcoreMesh(core_axis_name="c", subcore_axis_name="s")


def _tc_relayout(table, nrows, bt, cb):
    width = bt * 128

    def body(in_ref, out_ref):
        x = in_ref[...].reshape(8, bt, 128)
        for s in range(8):
            out_ref[:, s, :] = x[s]

    out3 = pl.pallas_call(
        body,
        grid=(K // 8, cb),
        in_specs=[pl.BlockSpec((8, width), lambda g, c: (g, c))],
        out_specs=pl.BlockSpec((bt, 8, 128), lambda g, c: (g * cb + c, 0, 0)),
        out_shape=jax.ShapeDtypeStruct(((K // 8) * cb * bt, 8, 128),
                                       jnp.float32),
    )(table)
    return jnp.reshape(out3, ((K // 8) * cb * bt * 1024,))


@functools.partial(
    pl.kernel,
    mesh=_mesh,
    out_type=jax.ShapeDtypeStruct((B,), jnp.float32),
    compiler_params=pltpu.CompilerParams(
        needs_layout_passes=False, use_tc_tiling_on_sc=False),
    scratch_types=[
        pltpu.VMEM((BPW,), jnp.int32),
        pltpu.VMEM((BPW,), jnp.int32),
        pltpu.VMEM((2 * EPW,), jnp.int32),
        pltpu.VMEM((2 * EPW,), jnp.int32),
        pltpu.VMEM((2 * EPW,), jnp.float32),
        pltpu.VMEM((2 * EPW,), jnp.float32),
        pltpu.VMEM((BPW,), jnp.float32),
        pltpu.SemaphoreType.DMA,
    ],
)
def _lookup(uidx_hbm, vidx_hbm, uflat_hbm, vflat_hbm, out_hbm,
            ubase, vbase, uoff, voff, uel, vel, outv, sem):
    wid = lax.axis_index("s") * NC + lax.axis_index("c")
    base = wid * BPW
    pltpu.sync_copy(uidx_hbm.at[pl.ds(base, BPW)], ubase)
    pltpu.sync_copy(vidx_hbm.at[pl.ds(base, BPW)], vbase)

    for g in range(NG):
        ru = ubase[pl.ds(g * RB, RB)]
        rv = vbase[pl.ds(g * RB, RB)]
        ubase[pl.ds(g * RB, RB)] = (ru >> 7) * 1024 + (ru & 127)
        vbase[pl.ds(g * RB, RB)] = (rv >> 7) * 1024 + (rv & 127)

    zeros16 = jnp.zeros((RB,), jnp.float32)
    for b in range(NG):
        outv[pl.ds(b * RB, RB)] = zeros16

    def dot_wave(half):
        for b in range(NG):
            acc = outv[pl.ds(b * RB, RB)]
            for kk in range(KPWAVE):
                acc = acc + (uel[pl.ds(half * EPW + kk * BPW + b * RB, RB)]
                             * vel[pl.ds(half * EPW + kk * BPW + b * RB, RB)])
            outv[pl.ds(b * RB, RB)] = acc

    def wave(w, _):
        half = w % 2
        hoff = half * EPW
        for kk in range(KPWAVE):
            k = w * KPWAVE + kk
            ukc = (k // 8) * UGS + (k % 8) * 128
            vkc = (k // 8) * VGS + (k % 8) * 128
            for g in range(NG):
                uoff[pl.ds(hoff + kk * BPW + g * RB, RB)] = (
                    ubase[pl.ds(g * RB, RB)] + ukc)
                voff[pl.ds(hoff + kk * BPW + g * RB, RB)] = (
                    vbase[pl.ds(g * RB, RB)] + vkc)
        for c0 in range(CPW):
            pltpu.async_copy(uflat_hbm.at[uoff.at[pl.ds(hoff + c0 * CH, CH)]],
                             uel.at[pl.ds(hoff + c0 * CH, CH)], sem)
            pltpu.async_copy(vflat_hbm.at[voff.at[pl.ds(hoff + c0 * CH, CH)]],
                             vel.at[pl.ds(hoff + c0 * CH, CH)], sem)

        @pl.when(w > 0)
        def _drain_and_reduce_prev():
            pltpu.make_async_copy(uflat_hbm.at[pl.ds(0, EPW)],
                                  uel.at[pl.ds(0, EPW)], sem).wait()
            pltpu.make_async_copy(vflat_hbm.at[pl.ds(0, EPW)],
                                  vel.at[pl.ds(0, EPW)], sem).wait()
            dot_wave(1 - half)
        return 0

    lax.fori_loop(0, NWAVE, wave, 0)
    pltpu.make_async_copy(uflat_hbm.at[pl.ds(0, EPW)],
                          uel.at[pl.ds(0, EPW)], sem).wait()
    pltpu.make_async_copy(vflat_hbm.at[pl.ds(0, EPW)],
                          vel.at[pl.ds(0, EPW)], sem).wait()
    dot_wave((NWAVE - 1) % 2)
    pltpu.sync_copy(outv, out_hbm.at[pl.ds(base, BPW)])


def kernel(user_index, item_index, Uemb, Vemb):
    uflat = _tc_relayout(Uemb.T, UROWS, UBT, UCB)
    vflat = _tc_relayout(Vemb.T, VROWS, VBT, VCB)
    return _lookup(user_index.astype(jnp.int32), item_index.astype(jnp.int32),
                   uflat, vflat)


# speedup vs baseline: 1.3331x; 1.3331x over previous
import functools

import jax
import jax.numpy as jnp
from jax import lax
from jax.experimental import pallas as pl
from jax.experimental.pallas import tpu as pltpu
from jax.experimental.pallas import tpu_sc as plsc

B = 16384
K = 32
NC = 2
NS = 16
NW = NC * NS
BPW = B // NW
RB = 16
NE = BPW * K
NG = BPW // RB
CH = 128
CPW = 8
EPW = CH * CPW
NWAVE = NE // EPW
KPWAVE = EPW // BPW

UROWS, VROWS = 1000000, 100000
UBT = 977
UCB = 8
VBT = 391
VCB = 2
UGS = UCB * UBT * 1024
VGS = VCB * VBT * 1024

_mesh = plsc.VectorSubcoreMesh(core_axis_name="c", subcore_axis_name="s")


def _tc_relayout(table, nrows, bt, cb):
    width = bt * 128

    def body(in_ref, out_ref):
        out_ref[...] = jnp.swapaxes(in_ref[...].reshape(8, bt, 128), 0, 1)

    out3 = pl.pallas_call(
        body,
        grid=(K // 8, cb),
        in_specs=[pl.BlockSpec((8, width), lambda g, c: (g, c))],
        out_specs=pl.BlockSpec((bt, 8, 128), lambda g, c: (g * cb + c, 0, 0)),
        out_shape=jax.ShapeDtypeStruct(((K // 8) * cb * bt, 8, 128),
                                       jnp.float32),
    )(table)
    return jnp.reshape(out3, ((K // 8) * cb * bt * 1024,))


@functools.partial(
    pl.kernel,
    mesh=_mesh,
    out_type=jax.ShapeDtypeStruct((B,), jnp.float32),
    compiler_params=pltpu.CompilerParams(
        needs_layout_passes=False, use_tc_tiling_on_sc=False),
    scratch_types=[
        pltpu.VMEM((BPW,), jnp.int32),
        pltpu.VMEM((BPW,), jnp.int32),
        pltpu.VMEM((2 * EPW,), jnp.int32),
        pltpu.VMEM((2 * EPW,), jnp.int32),
        pltpu.VMEM((2 * EPW,), jnp.float32),
        pltpu.VMEM((2 * EPW,), jnp.float32),
        pltpu.VMEM((BPW,), jnp.float32),
        pltpu.SemaphoreType.DMA,
    ],
)
def _lookup(uidx_hbm, vidx_hbm, uflat_hbm, vflat_hbm, out_hbm,
            ubase, vbase, uoff, voff, uel, vel, outv, sem):
    wid = lax.axis_index("s") * NC + lax.axis_index("c")
    base = wid * BPW
    pltpu.sync_copy(uidx_hbm.at[pl.ds(base, BPW)], ubase)
    pltpu.sync_copy(vidx_hbm.at[pl.ds(base, BPW)], vbase)

    for g in range(NG):
        ru = ubase[pl.ds(g * RB, RB)]
        rv = vbase[pl.ds(g * RB, RB)]
        ubase[pl.ds(g * RB, RB)] = (ru >> 7) * 1024 + (ru & 127)
        vbase[pl.ds(g * RB, RB)] = (rv >> 7) * 1024 + (rv & 127)

    zeros16 = jnp.zeros((RB,), jnp.float32)
    for b in range(NG):
        outv[pl.ds(b * RB, RB)] = zeros16

    def dot_wave(half):
        for b in range(NG):
            acc = outv[pl.ds(b * RB, RB)]
            for kk in range(KPWAVE):
                acc = acc + (uel[pl.ds(half * EPW + kk * BPW + b * RB, RB)]
                             * vel[pl.ds(half * EPW + kk * BPW + b * RB, RB)])
            outv[pl.ds(b * RB, RB)] = acc

    def wave(w, _):
        half = w % 2
        hoff = half * EPW
        for kk in range(KPWAVE):
            k = w * KPWAVE + kk
            ukc = (k // 8) * UGS + (k % 8) * 128
            vkc = (k // 8) * VGS + (k % 8) * 128
            for g in range(NG):
                uoff[pl.ds(hoff + kk * BPW + g * RB, RB)] = (
                    ubase[pl.ds(g * RB, RB)] + ukc)
                voff[pl.ds(hoff + kk * BPW + g * RB, RB)] = (
                    vbase[pl.ds(g * RB, RB)] + vkc)
        for c0 in range(CPW):
            pltpu.async_copy(uflat_hbm.at[uoff.at[pl.ds(hoff + c0 * CH, CH)]],
                             uel.at[pl.ds(hoff + c0 * CH, CH)], sem)
            pltpu.async_copy(vflat_hbm.at[voff.at[pl.ds(hoff + c0 * CH, CH)]],
                             vel.at[pl.ds(hoff + c0 * CH, CH)], sem)

        @pl.when(w > 0)
        def _drain_and_reduce_prev():
            pltpu.make_async_copy(uflat_hbm.at[pl.ds(0, EPW)],
                                  uel.at[pl.ds(0, EPW)], sem).wait()
            pltpu.make_async_copy(vflat_hbm.at[pl.ds(0, EPW)],
                                  vel.at[pl.ds(0, EPW)], sem).wait()
            dot_wave(1 - half)
        return 0

    lax.fori_loop(0, NWAVE, wave, 0)
    pltpu.make_async_copy(uflat_hbm.at[pl.ds(0, EPW)],
                          uel.at[pl.ds(0, EPW)], sem).wait()
    pltpu.make_async_copy(vflat_hbm.at[pl.ds(0, EPW)],
                          vel.at[pl.ds(0, EPW)], sem).wait()
    dot_wave((NWAVE - 1) % 2)
    pltpu.sync_copy(outv, out_hbm.at[pl.ds(base, BPW)])


def kernel(user_index, item_index, Uemb, Vemb):
    uflat = _tc_relayout(Uemb.T, UROWS, UBT, UCB)
    vflat = _tc_relayout(Vemb.T, VROWS, VBT, VCB)
    return _lookup(user_index.astype(jnp.int32), item_index.astype(jnp.int32),
                   uflat, vflat)


# speedup vs baseline: 1.4023x; 1.0519x over previous
import functools

import jax
import jax.numpy as jnp
from jax import lax
from jax.experimental import pallas as pl
from jax.experimental.pallas import tpu as pltpu
from jax.experimental.pallas import tpu_sc as plsc

B = 16384
K = 32
NC = 2
NS = 16
NW = NC * NS
BPW = B // NW
RB = 16
NE = BPW * K
NG = BPW // RB
CH = 128
CPW = 8
EPW = CH * CPW
NWAVE = NE // EPW
KPWAVE = EPW // BPW

UROWS, VROWS = 1000000, 100000
UBT = 977
UCB = 8
VBT = 391
VCB = 2
UGS = UCB * UBT * 1024
VGS = VCB * VBT * 1024

_mesh = plsc.VectorSubcoreMesh(core_axis_name="c", subcore_axis_name="s")


def _tc_relayout(table, nrows, bt, cb, g0=0, ngroups=K // 8):
    width = bt * 128

    def body(in_ref, out_ref):
        out_ref[...] = jnp.swapaxes(in_ref[...].reshape(8, bt, 128), 0, 1)

    out3 = pl.pallas_call(
        body,
        grid=(ngroups, cb),
        in_specs=[pl.BlockSpec((8, width), lambda g, c: (g + g0, c))],
        out_specs=pl.BlockSpec((bt, 8, 128), lambda g, c: (g * cb + c, 0, 0)),
        out_shape=jax.ShapeDtypeStruct((ngroups * cb * bt, 8, 128),
                                       jnp.float32),
    )(table)
    return jnp.reshape(out3, (ngroups * cb * bt * 1024,))


def _make_lookup(k0, nfeat):
    nwave = nfeat * BPW // EPW

    @functools.partial(
        pl.kernel,
        mesh=_mesh,
        out_type=jax.ShapeDtypeStruct((B,), jnp.float32),
        compiler_params=pltpu.CompilerParams(
            needs_layout_passes=False, use_tc_tiling_on_sc=False),
        scratch_types=[
            pltpu.VMEM((BPW,), jnp.int32),
            pltpu.VMEM((BPW,), jnp.int32),
            pltpu.VMEM((2 * EPW,), jnp.int32),
            pltpu.VMEM((2 * EPW,), jnp.int32),
            pltpu.VMEM((2 * EPW,), jnp.float32),
            pltpu.VMEM((2 * EPW,), jnp.float32),
            pltpu.VMEM((BPW,), jnp.float32),
            pltpu.SemaphoreType.DMA,
        ],
    )
    def _lookup(uidx_hbm, vidx_hbm, uflat_hbm, vflat_hbm, part_hbm, out_hbm,
                ubase, vbase, uoff, voff, uel, vel, outv, sem):
        wid = lax.axis_index("s") * NC + lax.axis_index("c")
        base = wid * BPW
        pltpu.sync_copy(uidx_hbm.at[pl.ds(base, BPW)], ubase)
        pltpu.sync_copy(vidx_hbm.at[pl.ds(base, BPW)], vbase)
        pltpu.sync_copy(part_hbm.at[pl.ds(base, BPW)], outv)

        for g in range(NG):
            ru = ubase[pl.ds(g * RB, RB)]
            rv = vbase[pl.ds(g * RB, RB)]
            ubase[pl.ds(g * RB, RB)] = (ru >> 7) * 1024 + (ru & 127)
            vbase[pl.ds(g * RB, RB)] = (rv >> 7) * 1024 + (rv & 127)

        def dot_wave(half):
            for b in range(NG):
                acc = outv[pl.ds(b * RB, RB)]
                for kk in range(KPWAVE):
                    acc = acc + (
                        uel[pl.ds(half * EPW + kk * BPW + b * RB, RB)]
                        * vel[pl.ds(half * EPW + kk * BPW + b * RB, RB)])
                outv[pl.ds(b * RB, RB)] = acc

        def wave(w, _):
            half = w % 2
            hoff = half * EPW
            for kk in range(KPWAVE):
                krel = w * KPWAVE + kk
                ukc = (krel // 8) * UGS + (krel % 8) * 128
                kv = k0 + krel
                vkc = (kv // 8) * VGS + (kv % 8) * 128
                for g in range(NG):
                    uoff[pl.ds(hoff + kk * BPW + g * RB, RB)] = (
                        ubase[pl.ds(g * RB, RB)] + ukc)
                    voff[pl.ds(hoff + kk * BPW + g * RB, RB)] = (
                        vbase[pl.ds(g * RB, RB)] + vkc)
            for c0 in range(CPW):
                pltpu.async_copy(
                    uflat_hbm.at[uoff.at[pl.ds(hoff + c0 * CH, CH)]],
                    uel.at[pl.ds(hoff + c0 * CH, CH)], sem)
                pltpu.async_copy(
                    vflat_hbm.at[voff.at[pl.ds(hoff + c0 * CH, CH)]],
                    vel.at[pl.ds(hoff + c0 * CH, CH)], sem)

            @pl.when(w > 0)
            def _drain_and_reduce_prev():
                pltpu.make_async_copy(uflat_hbm.at[pl.ds(0, EPW)],
                                      uel.at[pl.ds(0, EPW)], sem).wait()
                pltpu.make_async_copy(vflat_hbm.at[pl.ds(0, EPW)],
                                      vel.at[pl.ds(0, EPW)], sem).wait()
                dot_wave(1 - half)
            return 0

        lax.fori_loop(0, nwave, wave, 0)
        pltpu.make_async_copy(uflat_hbm.at[pl.ds(0, EPW)],
                              uel.at[pl.ds(0, EPW)], sem).wait()
        pltpu.make_async_copy(vflat_hbm.at[pl.ds(0, EPW)],
                              vel.at[pl.ds(0, EPW)], sem).wait()
        dot_wave((nwave - 1) % 2)
        pltpu.sync_copy(outv, out_hbm.at[pl.ds(base, BPW)])

    return _lookup


_lookup_a = _make_lookup(0, K // 2)
_lookup_b = _make_lookup(K // 2, K // 2)


def kernel(user_index, item_index, Uemb, Vemb):
    ui = user_index.astype(jnp.int32)
    vi = item_index.astype(jnp.int32)
    vflat = _tc_relayout(Vemb.T, VROWS, VBT, VCB)
    uflat_a = _tc_relayout(Uemb.T, UROWS, UBT, UCB, g0=0, ngroups=2)
    part = _lookup_a(ui, vi, uflat_a, vflat, jnp.zeros((B,), jnp.float32))
    uflat_b = _tc_relayout(Uemb.T, UROWS, UBT, UCB, g0=2, ngroups=2)
    return _lookup_b(ui, vi, uflat_b, vflat, part)
